# baseline (speedup 1.0000x reference)
import functools

import jax
import jax.numpy as jnp
from jax import lax
from jax.experimental import pallas as pl
from jax.experimental.pallas import tpu as pltpu
from jax.experimental.pallas import tpu_sc as plsc

N = 10000
E = 320000
F = 128
F3 = 40
F3P = 48

NC = 2
NS = 16
NW = NC * NS

CHUNK = 112
NCH = 90
EPT = NCH * CHUNK
EPAD = NW * EPT
RPT = 632
NP = NS * RPT

_f32 = jnp.float32
_i32 = jnp.int32

_MESH = plsc.VectorSubcoreMesh(core_axis_name="c", subcore_axis_name="s")


@functools.partial(
    pl.kernel,
    mesh=_MESH,
    compiler_params=pltpu.CompilerParams(use_tc_tiling_on_sc=False),
    out_type=jax.ShapeDtypeStruct((2 * NC * NP,), _f32),
    scratch_types=[
        pltpu.VMEM((NCH, CHUNK), _i32),
        pltpu.VMEM((NCH, CHUNK), _i32),
        pltpu.VMEM((CHUNK,), _f32),
        pltpu.VMEM((CHUNK,), _f32),
        pltpu.VMEM((RPT,), _f32),
        pltpu.VMEM_SHARED((NP,), _f32),
        pltpu.VMEM_SHARED((NP,), _f32),
        pltpu.SemaphoreType.DMA,
    ],
)
def _hist(row_hbm, col_hbm, out_hbm, rowv, colv, vew, vsf, zb, aew, asf, semi):
    cid = lax.axis_index("c")
    sid = lax.axis_index("s")
    wid = sid * NC + cid

    ir = pltpu.async_copy(row_hbm.at[wid], rowv, semi)
    ic = pltpu.async_copy(col_hbm.at[wid], colv, semi)

    z16 = jnp.zeros((16,), _f32)

    def zb_body(g, _):
        zb[pl.ds(g * 16, 16)] = z16
        return 0

    lax.fori_loop(0, RPT // 16, zb_body, 0)
    zb[pl.ds(RPT - 16, 16)] = z16
    pltpu.sync_copy(zb, aew.at[pl.ds(sid * RPT, RPT)])
    pltpu.sync_copy(zb, asf.at[pl.ds(sid * RPT, RPT)])
    ir.wait()
    ic.wait()
    plsc.subcore_barrier()

    def chunk(j, _):
        def grp(g, _):
            r = rowv[j, pl.ds(g * 16, 16)]
            c = colv[j, pl.ds(g * 16, 16)]
            ew = jnp.where(r != c, 1.0, 0.0).astype(_f32)
            vew[pl.ds(g * 16, 16)] = ew
            vsf[pl.ds(g * 16, 16)] = 1.0 - ew
            return 0

        lax.fori_loop(0, CHUNK // 16, grp, 0)
        pltpu.sync_copy(vew, aew.at[colv.at[j]], add=True)
        pltpu.sync_copy(vsf, asf.at[colv.at[j]], add=True)
        return 0

    lax.fori_loop(0, NCH, chunk, 0)
    plsc.subcore_barrier()
    pltpu.sync_copy(aew.at[pl.ds(sid * RPT, RPT)], zb)
    pltpu.sync_copy(zb, out_hbm.at[pl.ds(2 * cid * NP + sid * RPT, RPT)])
    pltpu.sync_copy(asf.at[pl.ds(sid * RPT, RPT)], zb)
    pltpu.sync_copy(zb, out_hbm.at[pl.ds((2 * cid + 1) * NP + sid * RPT, RPT)])


def _make_prop(feat):
    @functools.partial(
        pl.kernel,
        mesh=_MESH,
        compiler_params=pltpu.CompilerParams(use_tc_tiling_on_sc=False),
        out_type=jax.ShapeDtypeStruct((NC, NP, feat), _f32),
        scratch_types=[
            pltpu.VMEM((NCH + 2, CHUNK), _i32),
            pltpu.VMEM((NCH, CHUNK), _i32),
            pltpu.VMEM((CHUNK, feat), _f32),
            pltpu.VMEM((CHUNK, feat), _f32),
            pltpu.VMEM_SHARED((NP, feat), _f32),
            pltpu.SemaphoreType.DMA,
            pltpu.SemaphoreType.DMA,
            pltpu.SemaphoreType.DMA,
            pltpu.SemaphoreType.DMA,
        ],
    )
    def _prop(row_hbm, col_hbm, hs_hbm, out_hbm,
              rowv, colv, rows0, rows1, acc, semi, semg0, semg1, semz):
        cid = lax.axis_index("c")
        sid = lax.axis_index("s")
        wid = sid * NC + cid

        ir = pltpu.async_copy(row_hbm.at[wid], rowv.at[pl.ds(0, NCH)], semi)
        ic = pltpu.async_copy(col_hbm.at[wid], colv, semi)

        z16 = jnp.zeros((16,), _f32)
        for zr in range(CHUNK):
            for zc in range(feat // 16):
                rows0[zr, pl.ds(zc * 16, 16)] = z16
        nz = RPT // CHUNK
        zcopies = [pltpu.async_copy(
            rows0, acc.at[pl.ds(sid * RPT + k * CHUNK, CHUNK)], semz)
            for k in range(nz)]
        rem = RPT % CHUNK
        if rem:
            zcopies.append(pltpu.async_copy(
                rows0.at[pl.ds(0, rem)],
                acc.at[pl.ds(sid * RPT + RPT - rem, rem)], semz))
        ir.wait()
        z16i = jnp.zeros((16,), _i32)
        for zr in range(2):
            for zc in range(CHUNK // 16):
                rowv[NCH + zr, pl.ds(zc * 16, 16)] = z16i
        for c in zcopies:
            c.wait()
        ic.wait()
        plsc.subcore_barrier()

        pltpu.async_copy(hs_hbm.at[rowv.at[0]], rows0, semg0)
        pltpu.async_copy(hs_hbm.at[rowv.at[1]], rows1, semg1)
        hs_dummy = hs_hbm.at[pl.ds(0, CHUNK)]

        def pair(i, _):
            a = 2 * i
            pltpu.make_async_copy(hs_dummy, rows0, semg0).wait()
            pltpu.sync_copy(rows0, acc.at[colv.at[a]], add=True)
            pltpu.async_copy(hs_hbm.at[rowv.at[a + 2]], rows0, semg0)
            pltpu.make_async_copy(hs_dummy, rows1, semg1).wait()
            pltpu.sync_copy(rows1, acc.at[colv.at[a + 1]], add=True)
            pltpu.async_copy(hs_hbm.at[rowv.at[a + 3]], rows1, semg1)
            return 0

        lax.fori_loop(0, NCH // 2, pair, 0)
        pltpu.make_async_copy(hs_dummy, rows0, semg0).wait()
        pltpu.make_async_copy(hs_dummy, rows1, semg1).wait()

        plsc.subcore_barrier()
        for k in range(RPT // CHUNK):
            pltpu.sync_copy(acc.at[pl.ds(sid * RPT + k * CHUNK, CHUNK)], rows0)
            pltpu.sync_copy(rows0, out_hbm.at[cid, pl.ds(sid * RPT + k * CHUNK,
                                                         CHUNK)])
        if RPT % CHUNK:
            rem = RPT % CHUNK
            off = sid * RPT + RPT - rem
            pltpu.sync_copy(acc.at[pl.ds(off, rem)], rows0.at[pl.ds(0, rem)])
            pltpu.sync_copy(rows0.at[pl.ds(0, rem)], out_hbm.at[cid, pl.ds(off, rem)])

    return _prop


_prop128 = _make_prop(F)
_prop48 = _make_prop(F3P)


def _tc_prep_body(hist_ref, x_ref, w_ref, dis_ref, c0_ref, h_ref, hs_ref):
    hist = hist_ref[...]
    degn = (hist[0:N] + hist[2 * NP:2 * NP + N]).reshape(N, 1)
    selfc = (hist[NP:NP + N] + hist[3 * NP:3 * NP + N]).reshape(N, 1)
    dis = lax.rsqrt(degn + 1.0)
    c0 = (1.0 - selfc) * dis * dis
    h = jnp.dot(x_ref[...], w_ref[...], preferred_element_type=_f32)
    dis_ref[...] = dis
    c0_ref[...] = c0
    h_ref[...] = h
    hs_ref[...] = dis * h


_tc_prep = pl.pallas_call(
    _tc_prep_body,
    out_shape=[
        jax.ShapeDtypeStruct((N, 1), _f32),
        jax.ShapeDtypeStruct((N, 1), _f32),
        jax.ShapeDtypeStruct((N, F), _f32),
        jax.ShapeDtypeStruct((N, F), _f32),
    ],
)


def _tc_layer_body(p_ref, h_ref, dis_ref, c0_ref, b_ref, g_ref, be_ref,
                   w_ref, hn_ref, hsn_ref):
    p = p_ref[...]
    dis = dis_ref[...]
    u = (dis * (p[0, :N] + p[1, :N]) + c0_ref[...] * h_ref[...] + b_ref[...])
    mu = jnp.mean(u, axis=0, keepdims=True)
    d = u - mu
    var = jnp.mean(d * d, axis=0, keepdims=True)
    v = jnp.maximum(d * lax.rsqrt(var + 1e-5) * g_ref[...] + be_ref[...], 0.0)
    hn = jnp.dot(v, w_ref[...], preferred_element_type=_f32)
    hn_ref[...] = hn
    hsn_ref[...] = dis * hn


def _make_tc_layer(fout):
    return pl.pallas_call(
        _tc_layer_body,
        out_shape=[
            jax.ShapeDtypeStruct((N, fout), _f32),
            jax.ShapeDtypeStruct((N, fout), _f32),
        ],
    )


_tc_layer1 = _make_tc_layer(F)
_tc_layer2 = _make_tc_layer(F3P)


def _tc_final_body(p_ref, h_ref, dis_ref, c0_ref, b_ref, out_ref):
    p = p_ref[...]
    u = (dis_ref[...] * (p[0, :N] + p[1, :N])
         + c0_ref[...] * h_ref[...] + b_ref[...])
    out_ref[...] = u[:, :F3]


_tc_final = pl.pallas_call(
    _tc_final_body,
    out_shape=jax.ShapeDtypeStruct((N, F3), _f32),
)


def kernel(x, edge_idx, W1, b1, g1, be1, W2, b2, g2, be2, W3, b3):
    row = edge_idx[0]
    col = edge_idx[1]
    pad = EPAD - E
    rowp = jnp.concatenate([row, jnp.zeros((pad,), _i32)]).reshape(
        NW, NCH, CHUNK)
    colp = jnp.concatenate([col, jnp.full((pad,), N, _i32)]).reshape(
        NW, NCH, CHUNK)

    W3p = jnp.pad(W3, ((0, 0), (0, F3P - F3)))
    b3p = jnp.pad(b3, (0, F3P - F3)).reshape(1, F3P)

    hist = _hist(rowp, colp)
    dis, c0, h1, hs1 = _tc_prep(hist, x, W1)

    p1 = _prop128(rowp, colp, hs1)
    h2, hs2 = _tc_layer1(p1, h1, dis, c0, b1.reshape(1, F),
                         g1.reshape(1, F), be1.reshape(1, F), W2)

    p2 = _prop128(rowp, colp, hs2)
    h3, hs3 = _tc_layer2(p2, h2, dis, c0, b2.reshape(1, F),
                         g2.reshape(1, F), be2.reshape(1, F), W3p)

    p3 = _prop48(rowp, colp, hs3)
    return _tc_final(p3, h3, dis, c0, b3p)


# speedup vs baseline: 1.8945x; 1.8945x over previous
import functools

import jax
import jax.numpy as jnp
from jax import lax
from jax.experimental import pallas as pl
from jax.experimental.pallas import tpu as pltpu
from jax.experimental.pallas import tpu_sc as plsc

N = 10000
E = 320000
F = 128
F3 = 40
F3P = 64

NC = 2
NS = 16
NW = NC * NS

CHUNK = 128
NCH = 80
EPT = NCH * CHUNK
EPAD = NW * EPT
RPT = 632
NP = NS * RPT
H1 = 64
H3 = 32

_f32 = jnp.float32
_i32 = jnp.int32

_MESH = plsc.VectorSubcoreMesh(core_axis_name="c", subcore_axis_name="s")


@functools.partial(
    pl.kernel,
    mesh=_MESH,
    compiler_params=pltpu.CompilerParams(use_tc_tiling_on_sc=False),
    out_type=jax.ShapeDtypeStruct((2 * NC * NP,), _f32),
    scratch_types=[
        pltpu.VMEM((NCH, CHUNK), _i32),
        pltpu.VMEM((NCH, CHUNK), _i32),
        pltpu.VMEM((CHUNK,), _f32),
        pltpu.VMEM((CHUNK,), _f32),
        pltpu.VMEM((RPT,), _f32),
        pltpu.VMEM_SHARED((NP,), _f32),
        pltpu.VMEM_SHARED((NP,), _f32),
        pltpu.SemaphoreType.DMA,
    ],
)
def _hist(row_hbm, col_hbm, out_hbm, rowv, colv, vew, vsf, zb, aew, asf, semi):
    cid = lax.axis_index("c")
    sid = lax.axis_index("s")
    wid = sid * NC + cid

    ir = pltpu.async_copy(row_hbm.at[wid], rowv, semi)
    ic = pltpu.async_copy(col_hbm.at[wid], colv, semi)

    z16 = jnp.zeros((16,), _f32)

    def zb_body(g, _):
        zb[pl.ds(g * 16, 16)] = z16
        return 0

    lax.fori_loop(0, RPT // 16, zb_body, 0)
    zb[pl.ds(RPT - 16, 16)] = z16
    pltpu.sync_copy(zb, aew.at[pl.ds(sid * RPT, RPT)])
    pltpu.sync_copy(zb, asf.at[pl.ds(sid * RPT, RPT)])
    ir.wait()
    ic.wait()
    plsc.subcore_barrier()

    def chunk(j, _):
        def grp(g, _):
            r = rowv[j, pl.ds(g * 16, 16)]
            c = colv[j, pl.ds(g * 16, 16)]
            ew = jnp.where(r != c, 1.0, 0.0).astype(_f32)
            vew[pl.ds(g * 16, 16)] = ew
            vsf[pl.ds(g * 16, 16)] = 1.0 - ew
            return 0

        lax.fori_loop(0, CHUNK // 16, grp, 0)
        pltpu.sync_copy(vew, aew.at[colv.at[j]], add=True)
        pltpu.sync_copy(vsf, asf.at[colv.at[j]], add=True)
        return 0

    lax.fori_loop(0, NCH, chunk, 0)
    plsc.subcore_barrier()
    pltpu.sync_copy(aew.at[pl.ds(sid * RPT, RPT)], zb)
    pltpu.sync_copy(zb, out_hbm.at[pl.ds(2 * cid * NP + sid * RPT, RPT)])
    pltpu.sync_copy(asf.at[pl.ds(sid * RPT, RPT)], zb)
    pltpu.sync_copy(zb, out_hbm.at[pl.ds((2 * cid + 1) * NP + sid * RPT, RPT)])


def _make_prop(half):
    @functools.partial(
        pl.kernel,
        mesh=_MESH,
        compiler_params=pltpu.CompilerParams(use_tc_tiling_on_sc=False),
        out_type=jax.ShapeDtypeStruct((NC * NP, half), _f32),
        scratch_types=[
            pltpu.VMEM((2 * NCH, CHUNK), _i32),
            pltpu.VMEM((2 * NCH, CHUNK), _i32),
            pltpu.VMEM((CHUNK, half), _f32),
            pltpu.VMEM((8, half), _f32),
            pltpu.VMEM_SHARED((NP, half), _f32),
            pltpu.VMEM_SHARED((NP, half), _f32),
            pltpu.SemaphoreType.DMA,
            pltpu.SemaphoreType.DMA,
            pltpu.SemaphoreType.DMA,
            pltpu.SemaphoreType.DMA,
        ],
    )
    def _prop(row_hbm, col_hbm, hsa_hbm, hsb_hbm, out_hbm,
              rowv, colv, rows0, rows1, shs, acc, semi, semg0, semg1, semz):
        cid = lax.axis_index("c")
        sid = lax.axis_index("s")
        wid0 = sid * NC + cid
        wid1 = sid * NC + (1 - cid)

        ir0 = pltpu.async_copy(row_hbm.at[wid0], rowv.at[pl.ds(0, NCH)], semi)
        ic0 = pltpu.async_copy(col_hbm.at[wid0], colv.at[pl.ds(0, NCH)], semi)
        ir1 = pltpu.async_copy(row_hbm.at[wid1], rowv.at[pl.ds(NCH, NCH)],
                               semi)
        ic1 = pltpu.async_copy(col_hbm.at[wid1], colv.at[pl.ds(NCH, NCH)],
                               semi)

        z16 = jnp.zeros((16,), _f32)
        for zr in range(8):
            for zc in range(half // 16):
                rows1[zr, pl.ds(zc * 16, 16)] = z16
        zcopies = [pltpu.async_copy(
            rows1, acc.at[pl.ds(sid * RPT + k * 8, 8)], semz)
            for k in range(RPT // 8)]

        def stage(off, nrows):
            src = rows0.at[pl.ds(0, nrows)]

            @pl.when(cid == 0)
            def _():
                pltpu.sync_copy(hsa_hbm.at[pl.ds(off, nrows)], src)

            @pl.when(cid == 1)
            def _():
                pltpu.sync_copy(hsb_hbm.at[pl.ds(off, nrows)], src)

            pltpu.sync_copy(src, shs.at[pl.ds(off, nrows)])

        nf = RPT // CHUNK
        for k in range(nf):
            stage(sid * RPT + k * CHUNK, CHUNK)
        if RPT % CHUNK:
            stage(sid * RPT + nf * CHUNK, RPT % CHUNK)

        ir0.wait()
        ir1.wait()
        for c in zcopies:
            c.wait()
        ic0.wait()
        ic1.wait()
        plsc.subcore_barrier()

        def chunk(j, _):
            pltpu.async_copy(shs.at[rowv.at[j]], rows0, semg0).wait()
            pltpu.sync_copy(rows0, acc.at[colv.at[j]], add=True)
            return 0

        lax.fori_loop(0, 2 * NCH, chunk, 0)

        plsc.subcore_barrier()
        for k in range(RPT // CHUNK):
            off = sid * RPT + k * CHUNK
            pltpu.sync_copy(acc.at[pl.ds(off, CHUNK)], rows0)
            pltpu.sync_copy(rows0, out_hbm.at[pl.ds(cid * NP + off, CHUNK)])
        if RPT % CHUNK:
            rem = RPT % CHUNK
            off = sid * RPT + RPT - rem
            pltpu.sync_copy(acc.at[pl.ds(off, rem)], rows0.at[pl.ds(0, rem)])
            pltpu.sync_copy(rows0.at[pl.ds(0, rem)],
                            out_hbm.at[pl.ds(cid * NP + off, rem)])

    return _prop


_prop64 = _make_prop(H1)
_prop32 = _make_prop(H3)


def _tc_prep_body(hist_ref, x_ref, w_ref, dis_ref, c0_ref, h_ref,
                  hsa_ref, hsb_ref):
    hist = hist_ref[...]
    degn = (hist[0:N] + hist[2 * NP:2 * NP + N]).reshape(N, 1)
    selfc = (hist[NP:NP + N] + hist[3 * NP:3 * NP + N]).reshape(N, 1)
    dis = lax.rsqrt(degn + 1.0)
    c0 = (1.0 - selfc) * dis * dis
    h = jnp.dot(x_ref[...], w_ref[...], preferred_element_type=_f32)
    dis_ref[...] = dis
    c0_ref[...] = c0
    h_ref[...] = h
    hs = dis * h
    hsa_ref[:N, :] = hs[:, :H1]
    hsb_ref[:N, :] = hs[:, H1:]


_tc_prep = pl.pallas_call(
    _tc_prep_body,
    out_shape=[
        jax.ShapeDtypeStruct((N, 1), _f32),
        jax.ShapeDtypeStruct((N, 1), _f32),
        jax.ShapeDtypeStruct((N, F), _f32),
        jax.ShapeDtypeStruct((NP, H1), _f32),
        jax.ShapeDtypeStruct((NP, H1), _f32),
    ],
)


def _make_tc_layer(half_in, fout, half_out):
    def body(p_ref, h_ref, dis_ref, c0_ref, b_ref, g_ref, be_ref,
             w_ref, hn_ref, hsna_ref, hsnb_ref):
        dis = dis_ref[...]
        c0 = c0_ref[...]
        w = w_ref[...]
        parts = []
        for s in range(2):
            lo = s * half_in
            u = (dis * p_ref[s * NP:s * NP + N, :]
                 + c0 * h_ref[:, lo:lo + half_in]
                 + b_ref[:, lo:lo + half_in])
            mu = jnp.mean(u, axis=0, keepdims=True)
            d = u - mu
            var = jnp.mean(d * d, axis=0, keepdims=True)
            v = jnp.maximum(d * lax.rsqrt(var + 1e-5) * g_ref[:, lo:lo + half_in]
                            + be_ref[:, lo:lo + half_in], 0.0)
            parts.append(jnp.dot(v, w[lo:lo + half_in, :],
                                 preferred_element_type=_f32))
        hn = parts[0] + parts[1]
        hn_ref[...] = hn
        hsn = dis * hn
        hsna_ref[:N, :] = hsn[:, :half_out]
        hsnb_ref[:N, :] = hsn[:, half_out:]

    return pl.pallas_call(
        body,
        out_shape=[
            jax.ShapeDtypeStruct((N, fout), _f32),
            jax.ShapeDtypeStruct((NP, half_out), _f32),
            jax.ShapeDtypeStruct((NP, half_out), _f32),
        ],
    )


_tc_layer1 = _make_tc_layer(H1, F, H1)
_tc_layer2 = _make_tc_layer(H1, F3P, H3)


def _tc_final_body(p_ref, h_ref, dis_ref, c0_ref, b_ref, out_ref):
    dis = dis_ref[...]
    c0 = c0_ref[...]
    ua = dis * p_ref[:N, :] + c0 * h_ref[:, :H3] + b_ref[:, :H3]
    ub = dis * p_ref[NP:NP + N, :] + c0 * h_ref[:, H3:] + b_ref[:, H3:]
    out_ref[...] = jnp.concatenate([ua, ub[:, :F3 - H3]], axis=1)


_tc_final = pl.pallas_call(
    _tc_final_body,
    out_shape=jax.ShapeDtypeStruct((N, F3), _f32),
)


def kernel(x, edge_idx, W1, b1, g1, be1, W2, b2, g2, be2, W3, b3):
    row = edge_idx[0]
    col = edge_idx[1]
    pad = EPAD - E
    rowp = jnp.concatenate([row, jnp.zeros((pad,), _i32)]).reshape(
        NW, NCH, CHUNK)
    colp = jnp.concatenate([col, jnp.full((pad,), N, _i32)]).reshape(
        NW, NCH, CHUNK)

    W3p = jnp.pad(W3, ((0, 0), (0, F3P - F3)))
    b3p = jnp.pad(b3, (0, F3P - F3)).reshape(1, F3P)

    hist = _hist(rowp, colp)
    dis, c0, h1, hs1a, hs1b = _tc_prep(hist, x, W1)

    p1 = _prop64(rowp, colp, hs1a, hs1b)
    h2, hs2a, hs2b = _tc_layer1(p1, h1, dis, c0, b1.reshape(1, F),
                                g1.reshape(1, F), be1.reshape(1, F), W2)

    p2 = _prop64(rowp, colp, hs2a, hs2b)
    h3, hs3a, hs3b = _tc_layer2(p2, h2, dis, c0, b2.reshape(1, F),
                                g2.reshape(1, F), be2.reshape(1, F), W3p)

    p3 = _prop32(rowp, colp, hs3a, hs3b)
    return _tc_final(p3, h3, dis, c0, b3p)


# speedup vs baseline: 2.4198x; 1.2773x over previous
import functools

import jax
import jax.numpy as jnp
from jax import lax
from jax.experimental import pallas as pl
from jax.experimental.pallas import tpu as pltpu
from jax.experimental.pallas import tpu_sc as plsc

N = 10000
E = 320000
F = 128
F3 = 40
F3P = 64

NC = 2
NS = 16
NW = NC * NS

CHUNK = 128
NCH = 80
SLAB = 16
NSLAB = 2 * NCH // SLAB
EPT = NCH * CHUNK
EPAD = NW * EPT
RPT = 632
NP = NS * RPT
H1 = 64
H3 = 32

_f32 = jnp.float32
_i32 = jnp.int32

_MESH = plsc.VectorSubcoreMesh(core_axis_name="c", subcore_axis_name="s")


@functools.partial(
    pl.kernel,
    mesh=_MESH,
    compiler_params=pltpu.CompilerParams(use_tc_tiling_on_sc=False),
    out_type=jax.ShapeDtypeStruct((2 * NC * NP,), _f32),
    scratch_types=[
        pltpu.VMEM((NCH, CHUNK), _i32),
        pltpu.VMEM((NCH, CHUNK), _i32),
        pltpu.VMEM((CHUNK,), _f32),
        pltpu.VMEM((CHUNK,), _f32),
        pltpu.VMEM((RPT,), _f32),
        pltpu.VMEM_SHARED((NP,), _f32),
        pltpu.VMEM_SHARED((NP,), _f32),
        pltpu.SemaphoreType.DMA,
    ],
)
def _hist(row_hbm, col_hbm, out_hbm, rowv, colv, vew, vsf, zb, aew, asf, semi):
    cid = lax.axis_index("c")
    sid = lax.axis_index("s")
    wid = sid * NC + cid

    ir = pltpu.async_copy(row_hbm.at[wid], rowv, semi)
    ic = pltpu.async_copy(col_hbm.at[wid], colv, semi)

    z16 = jnp.zeros((16,), _f32)

    def zb_body(g, _):
        zb[pl.ds(g * 16, 16)] = z16
        return 0

    lax.fori_loop(0, RPT // 16, zb_body, 0)
    zb[pl.ds(RPT - 16, 16)] = z16
    pltpu.sync_copy(zb, aew.at[pl.ds(sid * RPT, RPT)])
    pltpu.sync_copy(zb, asf.at[pl.ds(sid * RPT, RPT)])
    ir.wait()
    ic.wait()
    plsc.subcore_barrier()

    def chunk(j, _):
        def grp(g, _):
            r = rowv[j, pl.ds(g * 16, 16)]
            c = colv[j, pl.ds(g * 16, 16)]
            ew = jnp.where(r != c, 1.0, 0.0).astype(_f32)
            vew[pl.ds(g * 16, 16)] = ew
            vsf[pl.ds(g * 16, 16)] = 1.0 - ew
            return 0

        lax.fori_loop(0, CHUNK // 16, grp, 0)
        pltpu.sync_copy(vew, aew.at[colv.at[j]], add=True)
        pltpu.sync_copy(vsf, asf.at[colv.at[j]], add=True)
        return 0

    lax.fori_loop(0, NCH, chunk, 0)
    plsc.subcore_barrier()
    pltpu.sync_copy(aew.at[pl.ds(sid * RPT, RPT)], zb)
    pltpu.sync_copy(zb, out_hbm.at[pl.ds(2 * cid * NP + sid * RPT, RPT)])
    pltpu.sync_copy(asf.at[pl.ds(sid * RPT, RPT)], zb)
    pltpu.sync_copy(zb, out_hbm.at[pl.ds((2 * cid + 1) * NP + sid * RPT, RPT)])


def _make_prop(half):
    @functools.partial(
        pl.kernel,
        mesh=_MESH,
        compiler_params=pltpu.CompilerParams(use_tc_tiling_on_sc=False),
        out_type=jax.ShapeDtypeStruct((NC * NP, half), _f32),
        scratch_types=[
            pltpu.VMEM((SLAB, CHUNK), _i32),
            pltpu.VMEM((SLAB, CHUNK), _i32),
            pltpu.VMEM((SLAB, CHUNK), _i32),
            pltpu.VMEM((SLAB, CHUNK), _i32),
            pltpu.VMEM((CHUNK, half), _f32),
            pltpu.VMEM((CHUNK, half), _f32),
            pltpu.VMEM((8, half), _f32),
            pltpu.VMEM_SHARED((NP, half), _f32),
            pltpu.VMEM_SHARED((NP, half), _f32),
            pltpu.SemaphoreType.DMA,
            pltpu.SemaphoreType.DMA,
            pltpu.SemaphoreType.DMA,
            pltpu.SemaphoreType.DMA,
            pltpu.SemaphoreType.DMA,
        ],
    )
    def _prop(row_hbm, col_hbm, hsa_hbm, hsb_hbm, out_hbm,
              r0r, r0c, r1r, r1c, gb0, gb1, zb, shs, acc,
              sidx0, sidx1, semg0, semg1, semz):
        cid = lax.axis_index("c")
        sid = lax.axis_index("s")
        wid0 = sid * NC + cid
        wid1 = sid * NC + (1 - cid)

        def fetch_slab(s, dr, dc, sem):
            t = wid0 if s < NSLAB // 2 else wid1
            k = (s % (NSLAB // 2)) * SLAB
            fr = pltpu.async_copy(row_hbm.at[t, pl.ds(k, SLAB)], dr, sem)
            fc = pltpu.async_copy(col_hbm.at[t, pl.ds(k, SLAB)], dc, sem)
            return fr, fc

        f0r, f0c = fetch_slab(0, r0r, r0c, sidx0)

        z16 = jnp.zeros((16,), _f32)
        for zr in range(8):
            for zc in range(half // 16):
                zb[zr, pl.ds(zc * 16, 16)] = z16
        zcopies = [pltpu.async_copy(
            zb, acc.at[pl.ds(sid * RPT + k * 8, 8)], semz)
            for k in range(RPT // 8)]

        def stage(off, nrows):
            src = gb0.at[pl.ds(0, nrows)]

            @pl.when(cid == 0)
            def _():
                pltpu.sync_copy(hsa_hbm.at[pl.ds(off, nrows)], src)

            @pl.when(cid == 1)
            def _():
                pltpu.sync_copy(hsb_hbm.at[pl.ds(off, nrows)], src)

            pltpu.sync_copy(src, shs.at[pl.ds(off, nrows)])

        nf = RPT // CHUNK
        for k in range(nf):
            stage(sid * RPT + k * CHUNK, CHUNK)
        if RPT % CHUNK:
            stage(sid * RPT + nf * CHUNK, RPT % CHUNK)

        for c in zcopies:
            c.wait()
        plsc.subcore_barrier()
        f0r.wait()
        f0c.wait()

        rings = [(r0r, r0c, sidx0), (r1r, r1c, sidx1)]
        gbufs = [gb0, gb1]
        gsems = [semg0, semg1]
        g = [pltpu.async_copy(shs.at[r0r.at[0]], gb0, semg0),
             pltpu.async_copy(shs.at[r0r.at[1]], gb1, semg1)]
        pend = [None, None]
        for s in range(NSLAB):
            tr, tc2, _ = rings[s % 2]
            if s + 1 < NSLAB:
                nr, ncol, nsem = rings[(s + 1) % 2]
                pend[(s + 1) % 2] = fetch_slab(s + 1, nr, ncol, nsem)
            for jj in range(SLAB):
                b = jj % 2
                g[b].wait()
                pltpu.sync_copy(gbufs[b], acc.at[tc2.at[jj]], add=True)
                if SLAB * s + jj + 2 < SLAB * NSLAB:
                    if jj == SLAB - 2:
                        fr, fc = pend[(s + 1) % 2]
                        fr.wait()
                        fc.wait()
                    if jj < SLAB - 2:
                        idx = tr.at[jj + 2]
                    else:
                        idx = rings[(s + 1) % 2][0].at[jj - (SLAB - 2)]
                    g[b] = pltpu.async_copy(shs.at[idx], gbufs[b], gsems[b])

        plsc.subcore_barrier()
        for k in range(RPT // CHUNK):
            off = sid * RPT + k * CHUNK
            pltpu.sync_copy(acc.at[pl.ds(off, CHUNK)], gb0)
            pltpu.sync_copy(gb0, out_hbm.at[pl.ds(cid * NP + off, CHUNK)])
        if RPT % CHUNK:
            rem = RPT % CHUNK
            off = sid * RPT + RPT - rem
            pltpu.sync_copy(acc.at[pl.ds(off, rem)], gb0.at[pl.ds(0, rem)])
            pltpu.sync_copy(gb0.at[pl.ds(0, rem)],
                            out_hbm.at[pl.ds(cid * NP + off, rem)])

    return _prop


_prop64 = _make_prop(H1)
_prop32 = _make_prop(H3)


def _tc_prep_body(hist_ref, x_ref, w_ref, dis_ref, c0_ref, h_ref,
                  hsa_ref, hsb_ref):
    hist = hist_ref[...]
    degn = (hist[0:N] + hist[2 * NP:2 * NP + N]).reshape(N, 1)
    selfc = (hist[NP:NP + N] + hist[3 * NP:3 * NP + N]).reshape(N, 1)
    dis = lax.rsqrt(degn + 1.0)
    c0 = (1.0 - selfc) * dis * dis
    h = jnp.dot(x_ref[...], w_ref[...], preferred_element_type=_f32)
    dis_ref[...] = dis
    c0_ref[...] = c0
    h_ref[...] = h
    hs = dis * h
    hsa_ref[:N, :] = hs[:, :H1]
    hsb_ref[:N, :] = hs[:, H1:]


_tc_prep = pl.pallas_call(
    _tc_prep_body,
    out_shape=[
        jax.ShapeDtypeStruct((N, 1), _f32),
        jax.ShapeDtypeStruct((N, 1), _f32),
        jax.ShapeDtypeStruct((N, F), _f32),
        jax.ShapeDtypeStruct((NP, H1), _f32),
        jax.ShapeDtypeStruct((NP, H1), _f32),
    ],
)


def _make_tc_layer(half_in, fout, half_out):
    def body(p_ref, h_ref, dis_ref, c0_ref, b_ref, g_ref, be_ref,
             w_ref, hn_ref, hsna_ref, hsnb_ref):
        dis = dis_ref[...]
        c0 = c0_ref[...]
        w = w_ref[...]
        parts = []
        for s in range(2):
            lo = s * half_in
            u = (dis * p_ref[s * NP:s * NP + N, :]
                 + c0 * h_ref[:, lo:lo + half_in]
                 + b_ref[:, lo:lo + half_in])
            mu = jnp.mean(u, axis=0, keepdims=True)
            d = u - mu
            var = jnp.mean(d * d, axis=0, keepdims=True)
            v = jnp.maximum(d * lax.rsqrt(var + 1e-5) * g_ref[:, lo:lo + half_in]
                            + be_ref[:, lo:lo + half_in], 0.0)
            parts.append(jnp.dot(v, w[lo:lo + half_in, :],
                                 preferred_element_type=_f32))
        hn = parts[0] + parts[1]
        hn_ref[...] = hn
        hsn = dis * hn
        hsna_ref[:N, :] = hsn[:, :half_out]
        hsnb_ref[:N, :] = hsn[:, half_out:]

    return pl.pallas_call(
        body,
        out_shape=[
            jax.ShapeDtypeStruct((N, fout), _f32),
            jax.ShapeDtypeStruct((NP, half_out), _f32),
            jax.ShapeDtypeStruct((NP, half_out), _f32),
        ],
    )


_tc_layer1 = _make_tc_layer(H1, F, H1)
_tc_layer2 = _make_tc_layer(H1, F3P, H3)


def _tc_final_body(p_ref, h_ref, dis_ref, c0_ref, b_ref, out_ref):
    dis = dis_ref[...]
    c0 = c0_ref[...]
    ua = dis * p_ref[:N, :] + c0 * h_ref[:, :H3] + b_ref[:, :H3]
    ub = dis * p_ref[NP:NP + N, :] + c0 * h_ref[:, H3:] + b_ref[:, H3:]
    out_ref[...] = jnp.concatenate([ua, ub[:, :F3 - H3]], axis=1)


_tc_final = pl.pallas_call(
    _tc_final_body,
    out_shape=jax.ShapeDtypeStruct((N, F3), _f32),
)


def kernel(x, edge_idx, W1, b1, g1, be1, W2, b2, g2, be2, W3, b3):
    row = edge_idx[0]
    col = edge_idx[1]
    pad = EPAD - E
    rowp = jnp.concatenate([row, jnp.zeros((pad,), _i32)]).reshape(
        NW, NCH, CHUNK)
    colp = jnp.concatenate([col, jnp.full((pad,), N, _i32)]).reshape(
        NW, NCH, CHUNK)

    W3p = jnp.pad(W3, ((0, 0), (0, F3P - F3)))
    b3p = jnp.pad(b3, (0, F3P - F3)).reshape(1, F3P)

    hist = _hist(rowp, colp)
    dis, c0, h1, hs1a, hs1b = _tc_prep(hist, x, W1)

    p1 = _prop64(rowp, colp, hs1a, hs1b)
    h2, hs2a, hs2b = _tc_layer1(p1, h1, dis, c0, b1.reshape(1, F),
                                g1.reshape(1, F), be1.reshape(1, F), W2)

    p2 = _prop64(rowp, colp, hs2a, hs2b)
    h3, hs3a, hs3b = _tc_layer2(p2, h2, dis, c0, b2.reshape(1, F),
                                g2.reshape(1, F), be2.reshape(1, F), W3p)

    p3 = _prop32(rowp, colp, hs3a, hs3b)
    return _tc_final(p3, h3, dis, c0, b3p)


# speedup vs baseline: 2.4661x; 1.0191x over previous
import functools

import jax
import jax.numpy as jnp
from jax import lax
from jax.experimental import pallas as pl
from jax.experimental.pallas import tpu as pltpu
from jax.experimental.pallas import tpu_sc as plsc

N = 10000
E = 320000
F = 128
F3 = 40
F3P = 64

NC = 2
NS = 16
NW = NC * NS

CHUNK = 128
NCH = 80
SLAB = 16
NSLAB = 2 * NCH // SLAB
EPT = NCH * CHUNK
EPAD = NW * EPT
RPT = 632
NP = NS * RPT
H1 = 64
H3 = 32

_f32 = jnp.float32
_i32 = jnp.int32

_MESH = plsc.VectorSubcoreMesh(core_axis_name="c", subcore_axis_name="s")


@functools.partial(
    pl.kernel,
    mesh=_MESH,
    compiler_params=pltpu.CompilerParams(use_tc_tiling_on_sc=False),
    out_type=jax.ShapeDtypeStruct((2 * NC * NP,), _f32),
    scratch_types=[
        pltpu.VMEM((NCH, CHUNK), _i32),
        pltpu.VMEM((NCH, CHUNK), _i32),
        pltpu.VMEM((CHUNK,), _f32),
        pltpu.VMEM((CHUNK,), _f32),
        pltpu.VMEM((RPT,), _f32),
        pltpu.VMEM_SHARED((NP,), _f32),
        pltpu.VMEM_SHARED((NP,), _f32),
        pltpu.SemaphoreType.DMA,
    ],
)
def _hist(row_hbm, col_hbm, out_hbm, rowv, colv, vew, vsf, zb, aew, asf, semi):
    cid = lax.axis_index("c")
    sid = lax.axis_index("s")
    wid = sid * NC + cid

    ir = pltpu.async_copy(row_hbm.at[wid], rowv, semi)
    ic = pltpu.async_copy(col_hbm.at[wid], colv, semi)

    z16 = jnp.zeros((16,), _f32)

    def zb_body(g, _):
        zb[pl.ds(g * 16, 16)] = z16
        return 0

    lax.fori_loop(0, RPT // 16, zb_body, 0)
    zb[pl.ds(RPT - 16, 16)] = z16
    pltpu.sync_copy(zb, aew.at[pl.ds(sid * RPT, RPT)])
    pltpu.sync_copy(zb, asf.at[pl.ds(sid * RPT, RPT)])
    ir.wait()
    ic.wait()
    plsc.subcore_barrier()

    def chunk(j, _):
        def grp(g, _):
            r = rowv[j, pl.ds(g * 16, 16)]
            c = colv[j, pl.ds(g * 16, 16)]
            ew = jnp.where(r != c, 1.0, 0.0).astype(_f32)
            vew[pl.ds(g * 16, 16)] = ew
            vsf[pl.ds(g * 16, 16)] = 1.0 - ew
            return 0

        lax.fori_loop(0, CHUNK // 16, grp, 0)
        pltpu.sync_copy(vew, aew.at[colv.at[j]], add=True)
        pltpu.sync_copy(vsf, asf.at[colv.at[j]], add=True)
        return 0

    lax.fori_loop(0, NCH, chunk, 0)
    plsc.subcore_barrier()
    pltpu.sync_copy(aew.at[pl.ds(sid * RPT, RPT)], zb)
    pltpu.sync_copy(zb, out_hbm.at[pl.ds(2 * cid * NP + sid * RPT, RPT)])
    pltpu.sync_copy(asf.at[pl.ds(sid * RPT, RPT)], zb)
    pltpu.sync_copy(zb, out_hbm.at[pl.ds((2 * cid + 1) * NP + sid * RPT, RPT)])


def _make_prop(half):
    @functools.partial(
        pl.kernel,
        mesh=_MESH,
        compiler_params=pltpu.CompilerParams(use_tc_tiling_on_sc=False),
        out_type=jax.ShapeDtypeStruct((NC * NP, half), _f32),
        scratch_types=[
            pltpu.VMEM((SLAB, CHUNK), _i32),
            pltpu.VMEM((SLAB, CHUNK), _i32),
            pltpu.VMEM((SLAB, CHUNK), _i32),
            pltpu.VMEM((SLAB, CHUNK), _i32),
            pltpu.VMEM((CHUNK, half), _f32),
            pltpu.VMEM((CHUNK, half), _f32),
            pltpu.VMEM((CHUNK, half), _f32),
            pltpu.VMEM((CHUNK, half), _f32),
            pltpu.VMEM((8, half), _f32),
            pltpu.VMEM_SHARED((NP, half), _f32),
            pltpu.VMEM_SHARED((NP, half), _f32),
            pltpu.SemaphoreType.DMA,
            pltpu.SemaphoreType.DMA,
            pltpu.SemaphoreType.DMA,
            pltpu.SemaphoreType.DMA,
            pltpu.SemaphoreType.DMA,
            pltpu.SemaphoreType.DMA,
            pltpu.SemaphoreType.DMA,
        ],
    )
    def _prop(row_hbm, col_hbm, hsa_hbm, hsb_hbm, out_hbm,
              r0r, r0c, r1r, r1c, gb0, gb1, gb2, gb3, zb, shs, acc,
              sidx0, sidx1, semg0, semg1, semg2, semg3, semz):
        cid = lax.axis_index("c")
        sid = lax.axis_index("s")
        wid0 = sid * NC + cid
        wid1 = sid * NC + (1 - cid)

        def fetch_slab(s, dr, dc, sem):
            t = wid0 if s < NSLAB // 2 else wid1
            k = (s % (NSLAB // 2)) * SLAB
            fr = pltpu.async_copy(row_hbm.at[t, pl.ds(k, SLAB)], dr, sem)
            fc = pltpu.async_copy(col_hbm.at[t, pl.ds(k, SLAB)], dc, sem)
            return fr, fc

        f0r, f0c = fetch_slab(0, r0r, r0c, sidx0)

        z16 = jnp.zeros((16,), _f32)
        for zr in range(8):
            for zc in range(half // 16):
                zb[zr, pl.ds(zc * 16, 16)] = z16
        zcopies = [pltpu.async_copy(
            zb, acc.at[pl.ds(sid * RPT + k * 8, 8)], semz)
            for k in range(RPT // 8)]

        def stage(off, nrows):
            src = gb0.at[pl.ds(0, nrows)]

            @pl.when(cid == 0)
            def _():
                pltpu.sync_copy(hsa_hbm.at[pl.ds(off, nrows)], src)

            @pl.when(cid == 1)
            def _():
                pltpu.sync_copy(hsb_hbm.at[pl.ds(off, nrows)], src)

            pltpu.sync_copy(src, shs.at[pl.ds(off, nrows)])

        nf = RPT // CHUNK
        for k in range(nf):
            stage(sid * RPT + k * CHUNK, CHUNK)
        if RPT % CHUNK:
            stage(sid * RPT + nf * CHUNK, RPT % CHUNK)

        for c in zcopies:
            c.wait()
        plsc.subcore_barrier()
        f0r.wait()
        f0c.wait()

        DEPTH = 4
        rings = [(r0r, r0c, sidx0), (r1r, r1c, sidx1)]
        gbufs = [gb0, gb1, gb2, gb3]
        gsems = [semg0, semg1, semg2, semg3]
        g = [pltpu.async_copy(shs.at[r0r.at[d]], gbufs[d], gsems[d])
             for d in range(DEPTH)]
        pend = [None, None]
        for s in range(NSLAB):
            tr, tc2, _ = rings[s % 2]
            if s + 1 < NSLAB:
                nr, ncol, nsem = rings[(s + 1) % 2]
                pend[(s + 1) % 2] = fetch_slab(s + 1, nr, ncol, nsem)
            for jj in range(SLAB):
                b = jj % DEPTH
                g[b].wait()
                pltpu.sync_copy(gbufs[b], acc.at[tc2.at[jj]], add=True)
                if SLAB * s + jj + DEPTH < SLAB * NSLAB:
                    if jj == SLAB - DEPTH:
                        fr, fc = pend[(s + 1) % 2]
                        fr.wait()
                        fc.wait()
                    if jj < SLAB - DEPTH:
                        idx = tr.at[jj + DEPTH]
                    else:
                        idx = rings[(s + 1) % 2][0].at[jj - (SLAB - DEPTH)]
                    g[b] = pltpu.async_copy(shs.at[idx], gbufs[b], gsems[b])

        plsc.subcore_barrier()
        for k in range(RPT // CHUNK):
            off = sid * RPT + k * CHUNK
            pltpu.sync_copy(acc.at[pl.ds(off, CHUNK)], gb0)
            pltpu.sync_copy(gb0, out_hbm.at[pl.ds(cid * NP + off, CHUNK)])
        if RPT % CHUNK:
            rem = RPT % CHUNK
            off = sid * RPT + RPT - rem
            pltpu.sync_copy(acc.at[pl.ds(off, rem)], gb0.at[pl.ds(0, rem)])
            pltpu.sync_copy(gb0.at[pl.ds(0, rem)],
                            out_hbm.at[pl.ds(cid * NP + off, rem)])

    return _prop


_prop64 = _make_prop(H1)
_prop32 = _make_prop(H3)


def _tc_prep_body(hist_ref, x_ref, w_ref, dis_ref, c0_ref, h_ref,
                  hsa_ref, hsb_ref):
    hist = hist_ref[...]
    degn = (hist[0:N] + hist[2 * NP:2 * NP + N]).reshape(N, 1)
    selfc = (hist[NP:NP + N] + hist[3 * NP:3 * NP + N]).reshape(N, 1)
    dis = lax.rsqrt(degn + 1.0)
    c0 = (1.0 - selfc) * dis * dis
    h = jnp.dot(x_ref[...], w_ref[...], preferred_element_type=_f32)
    dis_ref[...] = dis
    c0_ref[...] = c0
    h_ref[...] = h
    hs = dis * h
    hsa_ref[:N, :] = hs[:, :H1]
    hsb_ref[:N, :] = hs[:, H1:]


_tc_prep = pl.pallas_call(
    _tc_prep_body,
    out_shape=[
        jax.ShapeDtypeStruct((N, 1), _f32),
        jax.ShapeDtypeStruct((N, 1), _f32),
        jax.ShapeDtypeStruct((N, F), _f32),
        jax.ShapeDtypeStruct((NP, H1), _f32),
        jax.ShapeDtypeStruct((NP, H1), _f32),
    ],
)


def _make_tc_layer(half_in, fout, half_out):
    def body(p_ref, h_ref, dis_ref, c0_ref, b_ref, g_ref, be_ref,
             w_ref, hn_ref, hsna_ref, hsnb_ref):
        dis = dis_ref[...]
        c0 = c0_ref[...]
        w = w_ref[...]
        parts = []
        for s in range(2):
            lo = s * half_in
            u = (dis * p_ref[s * NP:s * NP + N, :]
                 + c0 * h_ref[:, lo:lo + half_in]
                 + b_ref[:, lo:lo + half_in])
            mu = jnp.mean(u, axis=0, keepdims=True)
            d = u - mu
            var = jnp.mean(d * d, axis=0, keepdims=True)
            v = jnp.maximum(d * lax.rsqrt(var + 1e-5) * g_ref[:, lo:lo + half_in]
                            + be_ref[:, lo:lo + half_in], 0.0)
            parts.append(jnp.dot(v, w[lo:lo + half_in, :],
                                 preferred_element_type=_f32))
        hn = parts[0] + parts[1]
        hn_ref[...] = hn
        hsn = dis * hn
        hsna_ref[:N, :] = hsn[:, :half_out]
        hsnb_ref[:N, :] = hsn[:, half_out:]

    return pl.pallas_call(
        body,
        out_shape=[
            jax.ShapeDtypeStruct((N, fout), _f32),
            jax.ShapeDtypeStruct((NP, half_out), _f32),
            jax.ShapeDtypeStruct((NP, half_out), _f32),
        ],
    )


_tc_layer1 = _make_tc_layer(H1, F, H1)
_tc_layer2 = _make_tc_layer(H1, F3P, H3)


def _tc_final_body(p_ref, h_ref, dis_ref, c0_ref, b_ref, out_ref):
    dis = dis_ref[...]
    c0 = c0_ref[...]
    ua = dis * p_ref[:N, :] + c0 * h_ref[:, :H3] + b_ref[:, :H3]
    ub = dis * p_ref[NP:NP + N, :] + c0 * h_ref[:, H3:] + b_ref[:, H3:]
    out_ref[...] = jnp.concatenate([ua, ub[:, :F3 - H3]], axis=1)


_tc_final = pl.pallas_call(
    _tc_final_body,
    out_shape=jax.ShapeDtypeStruct((N, F3), _f32),
)


def kernel(x, edge_idx, W1, b1, g1, be1, W2, b2, g2, be2, W3, b3):
    row = edge_idx[0]
    col = edge_idx[1]
    pad = EPAD - E
    rowp = jnp.concatenate([row, jnp.zeros((pad,), _i32)]).reshape(
        NW, NCH, CHUNK)
    colp = jnp.concatenate([col, jnp.full((pad,), N, _i32)]).reshape(
        NW, NCH, CHUNK)

    W3p = jnp.pad(W3, ((0, 0), (0, F3P - F3)))
    b3p = jnp.pad(b3, (0, F3P - F3)).reshape(1, F3P)

    hist = _hist(rowp, colp)
    dis, c0, h1, hs1a, hs1b = _tc_prep(hist, x, W1)

    p1 = _prop64(rowp, colp, hs1a, hs1b)
    h2, hs2a, hs2b = _tc_layer1(p1, h1, dis, c0, b1.reshape(1, F),
                                g1.reshape(1, F), be1.reshape(1, F), W2)

    p2 = _prop64(rowp, colp, hs2a, hs2b)
    h3, hs3a, hs3b = _tc_layer2(p2, h2, dis, c0, b2.reshape(1, F),
                                g2.reshape(1, F), be2.reshape(1, F), W3p)

    p3 = _prop32(rowp, colp, hs3a, hs3b)
    return _tc_final(p3, h3, dis, c0, b3p)


# speedup vs baseline: 2.5172x; 1.0207x over previous
import functools

import jax
import jax.numpy as jnp
from jax import lax
from jax.experimental import pallas as pl
from jax.experimental.pallas import tpu as pltpu
from jax.experimental.pallas import tpu_sc as plsc

N = 10000
E = 320000
F = 128
F3 = 40
F3P = 64

NC = 2
NS = 16
NW = NC * NS

CHUNK = 128
NCH = 80
SLAB = 16
NSLAB = 2 * NCH // SLAB
EPT = NCH * CHUNK
EPAD = NW * EPT
RPT = 632
NP = NS * RPT
H1 = 64
H3 = 32

_f32 = jnp.float32
_i32 = jnp.int32

_MESH = plsc.VectorSubcoreMesh(core_axis_name="c", subcore_axis_name="s")


@functools.partial(
    pl.kernel,
    mesh=_MESH,
    compiler_params=pltpu.CompilerParams(use_tc_tiling_on_sc=False),
    out_type=jax.ShapeDtypeStruct((2 * NC * NP,), _f32),
    scratch_types=[
        pltpu.VMEM((NCH, CHUNK), _i32),
        pltpu.VMEM((NCH, CHUNK), _i32),
        pltpu.VMEM((CHUNK,), _f32),
        pltpu.VMEM((CHUNK,), _f32),
        pltpu.VMEM((RPT,), _f32),
        pltpu.VMEM_SHARED((NP,), _f32),
        pltpu.VMEM_SHARED((NP,), _f32),
        pltpu.SemaphoreType.DMA,
    ],
)
def _hist(row_hbm, col_hbm, out_hbm, rowv, colv, vew, vsf, zb, aew, asf, semi):
    cid = lax.axis_index("c")
    sid = lax.axis_index("s")
    wid = sid * NC + cid

    ir = pltpu.async_copy(row_hbm.at[wid], rowv, semi)
    ic = pltpu.async_copy(col_hbm.at[wid], colv, semi)

    z16 = jnp.zeros((16,), _f32)

    def zb_body(g, _):
        zb[pl.ds(g * 16, 16)] = z16
        return 0

    lax.fori_loop(0, RPT // 16, zb_body, 0)
    zb[pl.ds(RPT - 16, 16)] = z16
    pltpu.sync_copy(zb, aew.at[pl.ds(sid * RPT, RPT)])
    pltpu.sync_copy(zb, asf.at[pl.ds(sid * RPT, RPT)])
    ir.wait()
    ic.wait()
    plsc.subcore_barrier()

    def chunk(j, _):
        def grp(g, _):
            r = rowv[j, pl.ds(g * 16, 16)]
            c = colv[j, pl.ds(g * 16, 16)]
            ew = jnp.where(r != c, 1.0, 0.0).astype(_f32)
            vew[pl.ds(g * 16, 16)] = ew
            vsf[pl.ds(g * 16, 16)] = 1.0 - ew
            return 0

        lax.fori_loop(0, CHUNK // 16, grp, 0)
        pltpu.sync_copy(vew, aew.at[colv.at[j]], add=True)
        pltpu.sync_copy(vsf, asf.at[colv.at[j]], add=True)
        return 0

    lax.fori_loop(0, NCH, chunk, 0)
    plsc.subcore_barrier()
    pltpu.sync_copy(aew.at[pl.ds(sid * RPT, RPT)], zb)
    pltpu.sync_copy(zb, out_hbm.at[pl.ds(2 * cid * NP + sid * RPT, RPT)])
    pltpu.sync_copy(asf.at[pl.ds(sid * RPT, RPT)], zb)
    pltpu.sync_copy(zb, out_hbm.at[pl.ds((2 * cid + 1) * NP + sid * RPT, RPT)])


def _make_prop(half):
    @functools.partial(
        pl.kernel,
        mesh=_MESH,
        compiler_params=pltpu.CompilerParams(use_tc_tiling_on_sc=False),
        out_type=jax.ShapeDtypeStruct((NC * NP, half), _f32),
        scratch_types=[
            pltpu.VMEM((SLAB, CHUNK), _i32),
            pltpu.VMEM((SLAB, CHUNK), _i32),
            pltpu.VMEM((SLAB, CHUNK), _i32),
            pltpu.VMEM((SLAB, CHUNK), _i32),
            pltpu.VMEM((CHUNK, half), _f32),
            pltpu.VMEM((CHUNK, half), _f32),
            pltpu.VMEM((CHUNK, half), _f32),
            pltpu.VMEM((CHUNK, half), _f32),
            pltpu.VMEM((8, half), _f32),
            pltpu.VMEM_SHARED((NP, half), _f32),
            pltpu.VMEM_SHARED((NP, half), _f32),
            pltpu.SemaphoreType.DMA,
            pltpu.SemaphoreType.DMA,
            pltpu.SemaphoreType.DMA,
            pltpu.SemaphoreType.DMA,
            pltpu.SemaphoreType.DMA,
            pltpu.SemaphoreType.DMA,
            pltpu.SemaphoreType.DMA,
        ],
    )
    def _prop(row_hbm, col_hbm, hsa_hbm, hsb_hbm, out_hbm,
              r0r, r0c, r1r, r1c, gb0, gb1, gb2, gb3, zb, shs, acc,
              sidx0, sidx1, semg0, semg1, semg2, semg3, semz):
        cid = lax.axis_index("c")
        sid = lax.axis_index("s")
        wid0 = sid * NC + cid
        wid1 = sid * NC + (1 - cid)

        def fetch_slab(s, dr, dc, sem):
            t = wid0 if s < NSLAB // 2 else wid1
            k = (s % (NSLAB // 2)) * SLAB
            fr = pltpu.async_copy(row_hbm.at[t, pl.ds(k, SLAB)], dr, sem)
            fc = pltpu.async_copy(col_hbm.at[t, pl.ds(k, SLAB)], dc, sem)
            return fr, fc

        f0r, f0c = fetch_slab(0, r0r, r0c, sidx0)

        z16 = jnp.zeros((16,), _f32)
        for zr in range(8):
            for zc in range(half // 16):
                zb[zr, pl.ds(zc * 16, 16)] = z16
        zcopies = [pltpu.async_copy(
            zb, acc.at[pl.ds(sid * RPT + k * 8, 8)], semz)
            for k in range(RPT // 8)]

        gbufs = [gb0, gb1, gb2, gb3]
        gsems = [semg0, semg1, semg2, semg3]
        nf = RPT // CHUNK
        rem = RPT % CHUNK
        blocks = [(sid * RPT + k * CHUNK, CHUNK) for k in range(nf)]
        if rem:
            blocks.append((sid * RPT + nf * CHUNK, rem))

        def hop1(k):
            off, nrows = blocks[k]
            src = gbufs[k % 4].at[pl.ds(0, nrows)]

            @pl.when(cid == 0)
            def _():
                pltpu.async_copy(
                    hsa_hbm.at[pl.ds(off, nrows)], src, gsems[k % 4])

            @pl.when(cid == 1)
            def _():
                pltpu.async_copy(
                    hsb_hbm.at[pl.ds(off, nrows)], src, gsems[k % 4])

        for k in range(min(4, len(blocks))):
            hop1(k)
        for k in range(len(blocks)):
            off, nrows = blocks[k]
            src = gbufs[k % 4].at[pl.ds(0, nrows)]
            pltpu.make_async_copy(
                hsa_hbm.at[pl.ds(off, nrows)], src, gsems[k % 4]).wait()
            pltpu.sync_copy(src, shs.at[pl.ds(off, nrows)])
            if k + 4 < len(blocks):
                hop1(k + 4)

        for c in zcopies:
            c.wait()
        plsc.subcore_barrier()
        f0r.wait()
        f0c.wait()

        DEPTH = 4
        rings = [(r0r, r0c, sidx0), (r1r, r1c, sidx1)]
        g = [pltpu.async_copy(shs.at[r0r.at[d]], gbufs[d], gsems[d])
             for d in range(DEPTH)]
        pend = [None, None]
        for s in range(NSLAB):
            tr, tc2, _ = rings[s % 2]
            if s + 1 < NSLAB:
                nr, ncol, nsem = rings[(s + 1) % 2]
                pend[(s + 1) % 2] = fetch_slab(s + 1, nr, ncol, nsem)
            for jj in range(SLAB):
                b = jj % DEPTH
                g[b].wait()
                pltpu.sync_copy(gbufs[b], acc.at[tc2.at[jj]], add=True)
                if SLAB * s + jj + DEPTH < SLAB * NSLAB:
                    if jj == SLAB - DEPTH:
                        fr, fc = pend[(s + 1) % 2]
                        fr.wait()
                        fc.wait()
                    if jj < SLAB - DEPTH:
                        idx = tr.at[jj + DEPTH]
                    else:
                        idx = rings[(s + 1) % 2][0].at[jj - (SLAB - DEPTH)]
                    g[b] = pltpu.async_copy(shs.at[idx], gbufs[b], gsems[b])

        plsc.subcore_barrier()
        def rhop1(k):
            off, nrows = blocks[k]
            return pltpu.async_copy(
                acc.at[pl.ds(off, nrows)],
                gbufs[k % 4].at[pl.ds(0, nrows)], gsems[k % 4])

        for k in range(min(4, len(blocks))):
            rhop1(k)
        for k in range(len(blocks)):
            off, nrows = blocks[k]
            src = gbufs[k % 4].at[pl.ds(0, nrows)]
            pltpu.make_async_copy(
                acc.at[pl.ds(off, nrows)], src, gsems[k % 4]).wait()
            pltpu.sync_copy(src, out_hbm.at[pl.ds(cid * NP + off, nrows)])
            if k + 4 < len(blocks):
                rhop1(k + 4)

    return _prop


_prop64 = _make_prop(H1)
_prop32 = _make_prop(H3)


def _tc_prep_body(hist_ref, x_ref, w_ref, dis_ref, c0_ref, h_ref,
                  hsa_ref, hsb_ref):
    hist = hist_ref[...]
    degn = (hist[0:N] + hist[2 * NP:2 * NP + N]).reshape(N, 1)
    selfc = (hist[NP:NP + N] + hist[3 * NP:3 * NP + N]).reshape(N, 1)
    dis = lax.rsqrt(degn + 1.0)
    c0 = (1.0 - selfc) * dis * dis
    h = jnp.dot(x_ref[...], w_ref[...], preferred_element_type=_f32)
    dis_ref[...] = dis
    c0_ref[...] = c0
    h_ref[...] = h
    hs = dis * h
    hsa_ref[:N, :] = hs[:, :H1]
    hsb_ref[:N, :] = hs[:, H1:]


_tc_prep = pl.pallas_call(
    _tc_prep_body,
    out_shape=[
        jax.ShapeDtypeStruct((N, 1), _f32),
        jax.ShapeDtypeStruct((N, 1), _f32),
        jax.ShapeDtypeStruct((N, F), _f32),
        jax.ShapeDtypeStruct((NP, H1), _f32),
        jax.ShapeDtypeStruct((NP, H1), _f32),
    ],
)


def _make_tc_layer(half_in, fout, half_out):
    def body(p_ref, h_ref, dis_ref, c0_ref, b_ref, g_ref, be_ref,
             w_ref, hn_ref, hsna_ref, hsnb_ref):
        dis = dis_ref[...]
        c0 = c0_ref[...]
        w = w_ref[...]
        parts = []
        for s in range(2):
            lo = s * half_in
            u = (dis * p_ref[s * NP:s * NP + N, :]
                 + c0 * h_ref[:, lo:lo + half_in]
                 + b_ref[:, lo:lo + half_in])
            mu = jnp.mean(u, axis=0, keepdims=True)
            d = u - mu
            var = jnp.mean(d * d, axis=0, keepdims=True)
            v = jnp.maximum(d * lax.rsqrt(var + 1e-5) * g_ref[:, lo:lo + half_in]
                            + be_ref[:, lo:lo + half_in], 0.0)
            parts.append(jnp.dot(v, w[lo:lo + half_in, :],
                                 preferred_element_type=_f32))
        hn = parts[0] + parts[1]
        hn_ref[...] = hn
        hsn = dis * hn
        hsna_ref[:N, :] = hsn[:, :half_out]
        hsnb_ref[:N, :] = hsn[:, half_out:]

    return pl.pallas_call(
        body,
        out_shape=[
            jax.ShapeDtypeStruct((N, fout), _f32),
            jax.ShapeDtypeStruct((NP, half_out), _f32),
            jax.ShapeDtypeStruct((NP, half_out), _f32),
        ],
    )


_tc_layer1 = _make_tc_layer(H1, F, H1)
_tc_layer2 = _make_tc_layer(H1, F3P, H3)


def _tc_final_body(p_ref, h_ref, dis_ref, c0_ref, b_ref, out_ref):
    dis = dis_ref[...]
    c0 = c0_ref[...]
    ua = dis * p_ref[:N, :] + c0 * h_ref[:, :H3] + b_ref[:, :H3]
    ub = dis * p_ref[NP:NP + N, :] + c0 * h_ref[:, H3:] + b_ref[:, H3:]
    out_ref[...] = jnp.concatenate([ua, ub[:, :F3 - H3]], axis=1)


_tc_final = pl.pallas_call(
    _tc_final_body,
    out_shape=jax.ShapeDtypeStruct((N, F3), _f32),
)


def kernel(x, edge_idx, W1, b1, g1, be1, W2, b2, g2, be2, W3, b3):
    row = edge_idx[0]
    col = edge_idx[1]
    pad = EPAD - E
    rowp = jnp.concatenate([row, jnp.zeros((pad,), _i32)]).reshape(
        NW, NCH, CHUNK)
    colp = jnp.concatenate([col, jnp.full((pad,), N, _i32)]).reshape(
        NW, NCH, CHUNK)

    W3p = jnp.pad(W3, ((0, 0), (0, F3P - F3)))
    b3p = jnp.pad(b3, (0, F3P - F3)).reshape(1, F3P)

    hist = _hist(rowp, colp)
    dis, c0, h1, hs1a, hs1b = _tc_prep(hist, x, W1)

    p1 = _prop64(rowp, colp, hs1a, hs1b)
    h2, hs2a, hs2b = _tc_layer1(p1, h1, dis, c0, b1.reshape(1, F),
                                g1.reshape(1, F), be1.reshape(1, F), W2)

    p2 = _prop64(rowp, colp, hs2a, hs2b)
    h3, hs3a, hs3b = _tc_layer2(p2, h2, dis, c0, b2.reshape(1, F),
                                g2.reshape(1, F), be2.reshape(1, F), W3p)

    p3 = _prop32(rowp, colp, hs3a, hs3b)
    return _tc_final(p3, h3, dis, c0, b3p)
